# Initial kernel scaffold; baseline (speedup 1.0000x reference)
import jax
import jax.numpy as jnp
from jax.experimental import pallas as pl


def kernel(h_V_s, h_V_v, edge_index, h_E_s, h_E_v, seq, params):
    raise NotImplementedError("write your pallas kernel here")


# speedup vs baseline: 6.8211x; 6.8211x over previous
import functools

import jax
import jax.numpy as jnp
from jax import lax
from jax.experimental import pallas as pl
from jax.experimental.pallas import tpu as pltpu
from jax.experimental.pallas import tpu_sc as plsc

N_N = 10000
N_E = 160000
ROW = 160
NC, NS = 2, 16
NW = NC * NS
CH = 128
NCH = 40
E_PAD = NW * NCH * CH
N_PAD = 10240
BE = 2048
BN = 2000
EPS = 1e-8


def _sc_mesh():
    return plsc.VectorSubcoreMesh(core_axis_name="c", subcore_axis_name="s")


def _sc_cp():
    return pltpu.CompilerParams(use_tc_tiling_on_sc=False)


def _sc_gather2(tab, idxj2, idxi2, D, dtype):
    out_t = jax.ShapeDtypeStruct((E_PAD, D), dtype)

    @functools.partial(
        pl.kernel,
        out_type=[out_t, out_t],
        mesh=_sc_mesh(),
        compiler_params=_sc_cp(),
        scratch_types=[
            pltpu.VMEM((NCH, CH), jnp.int32),
            pltpu.VMEM((NCH, CH), jnp.int32),
            pltpu.VMEM((CH, D), dtype),
            pltpu.VMEM((CH, D), dtype),
            pltpu.SemaphoreType.DMA,
            pltpu.SemaphoreType.DMA,
        ],
    )
    def k(tab_h, ij_h, ii_h, oj_h, oi_h, ijv, iiv, bj, bi, semj, semi):
        wid = lax.axis_index("s") * NC + lax.axis_index("c")
        c0 = wid * NCH
        pltpu.sync_copy(ij_h.at[pl.ds(c0, NCH)], ijv)
        pltpu.sync_copy(ii_h.at[pl.ds(c0, NCH)], iiv)

        @pl.loop(0, NCH)
        def _(t):
            cj = pltpu.async_copy(tab_h.at[ijv.at[t]], bj, semj)
            ci = pltpu.async_copy(tab_h.at[iiv.at[t]], bi, semi)
            cj.wait()
            base = (c0 + t) * CH
            pltpu.sync_copy(bj, oj_h.at[pl.ds(base, CH)])
            ci.wait()
            pltpu.sync_copy(bi, oi_h.at[pl.ds(base, CH)])

    return k(tab, idxj2, idxi2)


def _sc_gather1(tab, idx2, D, dtype):
    out_t = jax.ShapeDtypeStruct((E_PAD, D), dtype)

    @functools.partial(
        pl.kernel,
        out_type=out_t,
        mesh=_sc_mesh(),
        compiler_params=_sc_cp(),
        scratch_types=[
            pltpu.VMEM((NCH, CH), jnp.int32),
            pltpu.VMEM((CH, D), dtype),
            pltpu.SemaphoreType.DMA,
        ],
    )
    def k(tab_h, ix_h, o_h, ixv, buf, sem):
        wid = lax.axis_index("s") * NC + lax.axis_index("c")
        c0 = wid * NCH
        pltpu.sync_copy(ix_h.at[pl.ds(c0, NCH)], ixv)

        @pl.loop(0, NCH)
        def _(t):
            pltpu.async_copy(tab_h.at[ixv.at[t]], buf, sem).wait()
            pltpu.sync_copy(buf, o_h.at[pl.ds((c0 + t) * CH, CH)])

    return k(tab, idx2)


def _sc_scatter_add(vals, idx2, zeros, D):
    out_t = jax.ShapeDtypeStruct((N_PAD, D), jnp.float32)
    rows_per_sub = N_PAD // NS

    @functools.partial(
        pl.kernel,
        out_type=[out_t, out_t],
        mesh=_sc_mesh(),
        compiler_params=_sc_cp(),
        scratch_types=[
            pltpu.VMEM((NCH, CH), jnp.int32),
            pltpu.VMEM((CH, D), jnp.float32),
            pltpu.VMEM_SHARED((N_PAD, D), jnp.float32),
            pltpu.SemaphoreType.DMA,
        ],
    )
    def k(v_h, ix_h, z_h, o0_h, o1_h, ixv, buf, acc, sem):
        cid = lax.axis_index("c")
        sid = lax.axis_index("s")
        wid = sid * NC + cid
        c0 = wid * NCH
        pltpu.sync_copy(ix_h.at[pl.ds(c0, NCH)], ixv)
        r0 = sid * rows_per_sub
        pltpu.sync_copy(z_h.at[pl.ds(r0, rows_per_sub)],
                        acc.at[pl.ds(r0, rows_per_sub)])
        plsc.subcore_barrier()

        @pl.loop(0, NCH)
        def _(t):
            pltpu.async_copy(v_h.at[pl.ds((c0 + t) * CH, CH)], buf, sem).wait()
            pltpu.sync_copy(buf, acc.at[ixv.at[t]], add=True)

        plsc.subcore_barrier()

        @pl.when(cid == 0)
        def _():
            pltpu.sync_copy(acc.at[pl.ds(r0, rows_per_sub)],
                            o0_h.at[pl.ds(r0, rows_per_sub)])

        @pl.when(cid == 1)
        def _():
            pltpu.sync_copy(acc.at[pl.ds(r0, rows_per_sub)],
                            o1_h.at[pl.ds(r0, rows_per_sub)])

    return k(vals, idx2, zeros)


def _full(shape):
    return pl.BlockSpec(shape, lambda i: (0,) * len(shape))


def _cp():
    return pltpu.CompilerParams(dimension_semantics=("arbitrary",))


def _vslices(t):
    return [t[:, 100 + 16 * c:116 + 16 * c] for c in range(3)]


def _vec_gate(vo):
    nrm = jnp.sqrt(jnp.maximum(vo[0] * vo[0] + vo[1] * vo[1] + vo[2] * vo[2],
                               EPS))
    g = jax.nn.sigmoid(nrm)
    return [x * g for x in vo]


def _scalar_ln(s, g, b):
    mu = jnp.mean(s, axis=-1, keepdims=True)
    var = jnp.mean(jnp.square(s - mu), axis=-1, keepdims=True)
    return (s - mu) / jnp.sqrt(var + 1e-5) * g + b


def _vector_ln(v):
    nsq = jnp.maximum(v[0] * v[0] + v[1] * v[1] + v[2] * v[2], EPS)
    vn = jnp.sqrt(jnp.mean(nsq, axis=-1, keepdims=True))
    return [x / vn for x in v]


def _store_packed(o_ref, s, v, extra):
    o_ref[:, 0:100] = s
    for c in range(3):
        o_ref[:, 100 + 16 * c:116 + 16 * c] = v[c]
    o_ref[:, 148:160] = extra


def _node_embed(h_V_s, hvt, p0c, p1c, wh, ws_s, ws_n, b, wv, g, bb):
    def body(hvs_r, hvt_r, p0_r, p1_r, wh_r, wss_r, wsn_r, b_r, wv_r, g_r,
             bb_r, o_r):
        hvs, hvt_ = hvs_r[...], hvt_r[...]
        vh = [jnp.dot(hvt_[:, 3 * c:3 * c + 3], wh_r[...],
                      preferred_element_type=jnp.float32) for c in range(3)]
        vn = jnp.sqrt(jnp.maximum(vh[0] ** 2 + vh[1] ** 2 + vh[2] ** 2, EPS))
        s = (jnp.dot(hvs, wss_r[...], preferred_element_type=jnp.float32)
             + jnp.dot(vn, wsn_r[...], preferred_element_type=jnp.float32)
             + b_r[...])
        v = [jnp.dot(x, wv_r[...], preferred_element_type=jnp.float32)
             for x in vh]
        s = _scalar_ln(s, g_r[...], bb_r[...])
        v = _vector_ln(v)
        cnt = p0_r[:, 0:1] + p1_r[:, 0:1]
        cinv = 1.0 / jnp.maximum(cnt, 1.0)
        extra = jnp.concatenate(
            [cinv, jnp.zeros((cinv.shape[0], 11), jnp.float32)], axis=1)
        _store_packed(o_r, s, v, extra)

    grid = N_N // BN
    return pl.pallas_call(
        body,
        grid=(grid,),
        in_specs=[
            pl.BlockSpec((BN, 6), lambda i: (i, 0)),
            pl.BlockSpec((BN, 9), lambda i: (i, 0)),
            pl.BlockSpec((BN, 16), lambda i: (i, 0)),
            pl.BlockSpec((BN, 16), lambda i: (i, 0)),
            _full((3, 16)), _full((6, 100)), _full((16, 100)), _full((1, 100)),
            _full((16, 16)), _full((1, 100)), _full((1, 100)),
        ],
        out_specs=pl.BlockSpec((BN, ROW), lambda i: (i, 0)),
        out_shape=jax.ShapeDtypeStruct((N_N, ROW), jnp.float32),
        compiler_params=_cp(),
    )(h_V_s, hvt, p0c, p1c, wh, ws_s, ws_n, b, wv, g, bb)


def _edge_embed(hes, hev, ws_s, ws_n, b, wh00, wv00, g, bb):
    def body(hes_r, hev_r, wss_r, wsn_r, b_r, wh_r, wv_r, g_r, bb_r, o_r):
        es = hes_r[...]
        ev = [hev_r[:, c:c + 1] for c in range(3)]
        wh = wh_r[0, 0]
        vh = [x * wh for x in ev]
        vn = jnp.sqrt(jnp.maximum(vh[0] ** 2 + vh[1] ** 2 + vh[2] ** 2, EPS))
        s = (jnp.dot(es, wss_r[...], preferred_element_type=jnp.float32)
             + vn * wsn_r[...] + b_r[...])
        v = [x * wv_r[0, 0] for x in vh]
        s = _scalar_ln(s, g_r[...], bb_r[...])
        v = _vector_ln(v)
        o_r[:, 0:32] = s
        for c in range(3):
            o_r[:, 32 + c:33 + c] = v[c]
        o_r[:, 35:64] = jnp.zeros((s.shape[0], 29), jnp.float32)

    grid = E_PAD // BE
    return pl.pallas_call(
        body,
        grid=(grid,),
        in_specs=[
            pl.BlockSpec((BE, 32), lambda i: (i, 0)),
            pl.BlockSpec((BE, 8), lambda i: (i, 0)),
            _full((32, 32)), _full((1, 32)), _full((1, 32)),
            _full((1, 1)), _full((1, 1)), _full((1, 32)), _full((1, 32)),
        ],
        out_specs=pl.BlockSpec((BE, 64), lambda i: (i, 0)),
        out_shape=jax.ShapeDtypeStruct((E_PAD, 64), jnp.float32),
        compiler_params=_cp(),
    )(hes, hev, ws_s, ws_n, b, wh00, wv00, g, bb)


def _efdec_build(ef, hsg):
    def body(ef_r, hs_r, o_r):
        o_r[:, 0:32] = ef_r[:, 0:32]
        o_r[:, 32:52] = hs_r[:, 0:20]
        o_r[:, 52:55] = ef_r[:, 32:35]
        o_r[:, 55:64] = jnp.zeros((ef_r.shape[0], 9), jnp.float32)

    grid = E_PAD // BE
    return pl.pallas_call(
        body,
        grid=(grid,),
        in_specs=[pl.BlockSpec((BE, 64), lambda i: (i, 0)),
                  pl.BlockSpec((BE, 32), lambda i: (i, 0))],
        out_specs=pl.BlockSpec((BE, 64), lambda i: (i, 0)),
        out_shape=jax.ShapeDtypeStruct((E_PAD, 64), jnp.float32),
        compiler_params=_cp(),
    )(ef, hsg)


def _msg_mlp(gj, gi, ef, se, w):
    def body(gj_r, gi_r, ef_r, whj_r, whe_r, whi_r, w1j_r, w1e_r, w1i_r,
             w1n_r, b1_r, wv1_r, wh2_r, w2s_r, w2n_r, b2_r, wv2_r, wh3_r,
             w3s_r, w3n_r, b3_r, wv3_r, o_r):
        bid = pl.program_id(0)
        gjv, giv = gj_r[...], gi_r[...]
        efv = ef_r[...]
        gjs, gis = gjv[:, 0:100], giv[:, 0:100]
        es = efv[:, 0:se]

        dot = functools.partial(jnp.dot, preferred_element_type=jnp.float32)
        vh = []
        for c in range(3):
            mj = gjv[:, 100 + 16 * c:116 + 16 * c]
            mi = giv[:, 100 + 16 * c:116 + 16 * c]
            ev = efv[:, se + c:se + c + 1]
            vh.append(dot(mj, whj_r[...]) + ev * whe_r[...]
                      + dot(mi, whi_r[...]))
        vn = jnp.sqrt(jnp.maximum(vh[0] ** 2 + vh[1] ** 2 + vh[2] ** 2, EPS))
        s = (dot(gjs, w1j_r[...]) + dot(es, w1e_r[...]) + dot(gis, w1i_r[...])
             + dot(vn, w1n_r[...]) + b1_r[...])
        s = jnp.maximum(s, 0.0)
        v = _vec_gate([dot(x, wv1_r[...]) for x in vh])

        vh = [dot(x, wh2_r[...]) for x in v]
        vn = jnp.sqrt(jnp.maximum(vh[0] ** 2 + vh[1] ** 2 + vh[2] ** 2, EPS))
        s = jnp.maximum(dot(s, w2s_r[...]) + dot(vn, w2n_r[...]) + b2_r[...],
                        0.0)
        v = _vec_gate([dot(x, wv2_r[...]) for x in vh])

        vh = [dot(x, wh3_r[...]) for x in v]
        vn = jnp.sqrt(jnp.maximum(vh[0] ** 2 + vh[1] ** 2 + vh[2] ** 2, EPS))
        s = dot(s, w3s_r[...]) + dot(vn, w3n_r[...]) + b3_r[...]
        v = [dot(x, wv3_r[...]) for x in vh]

        rows = jax.lax.broadcasted_iota(jnp.int32, (gjv.shape[0], 1), 0)
        valid = (bid * BE + rows) < N_E
        ci = jnp.where(valid, giv[:, 148:149], 0.0)
        s = s * ci
        v = [x * ci for x in v]
        _store_packed(o_r, s, v, jnp.zeros((gjv.shape[0], 12), jnp.float32))

    grid = E_PAD // BE
    espec = [
        pl.BlockSpec((BE, ROW), lambda i: (i, 0)),
        pl.BlockSpec((BE, ROW), lambda i: (i, 0)),
        pl.BlockSpec((BE, 64), lambda i: (i, 0)),
    ]
    wspec = [_full(x.shape) for x in w]
    return pl.pallas_call(
        body,
        grid=(grid,),
        in_specs=espec + wspec,
        out_specs=pl.BlockSpec((BE, ROW), lambda i: (i, 0)),
        out_shape=jax.ShapeDtypeStruct((E_PAD, ROW), jnp.float32),
        compiler_params=_cp(),
    )(gj, gi, ef, *w)


def _node_update(t, p0, p1, w):
    def body(t_r, p0_r, p1_r, g0_r, b0_r, f1wh_r, f1ws_r, f1wn_r, f1b_r,
             f1wv_r, f2wh_r, f2ws_r, f2wn_r, f2b_r, f2wv_r, g1_r, b1_r, o_r):
        dot = functools.partial(jnp.dot, preferred_element_type=jnp.float32)
        tv = t_r[...]
        d0, d1 = p0_r[...], p1_r[...]
        dsum = d0 + d1
        s = tv[:, 0:100] + dsum[:, 0:100]
        v = [a + b for a, b in zip(_vslices(tv), _vslices(dsum))]
        s = _scalar_ln(s, g0_r[...], b0_r[...])
        v = _vector_ln(v)

        vh = [dot(x, f1wh_r[...]) for x in v]
        vn = jnp.sqrt(jnp.maximum(vh[0] ** 2 + vh[1] ** 2 + vh[2] ** 2, EPS))
        fs = jnp.maximum(dot(s, f1ws_r[...]) + dot(vn, f1wn_r[...])
                         + f1b_r[...], 0.0)
        fv = _vec_gate([dot(x, f1wv_r[...]) for x in vh])

        vh = [dot(x, f2wh_r[...]) for x in fv]
        vn = jnp.sqrt(jnp.maximum(vh[0] ** 2 + vh[1] ** 2 + vh[2] ** 2, EPS))
        fs = dot(fs, f2ws_r[...]) + dot(vn, f2wn_r[...]) + f2b_r[...]
        fv = [dot(x, f2wv_r[...]) for x in vh]

        s = s + fs
        v = [a + b for a, b in zip(v, fv)]
        s = _scalar_ln(s, g1_r[...], b1_r[...])
        v = _vector_ln(v)
        extra = jnp.concatenate(
            [tv[:, 148:149], jnp.zeros((tv.shape[0], 11), jnp.float32)],
            axis=1)
        _store_packed(o_r, s, v, extra)

    grid = N_N // BN
    nspec = [pl.BlockSpec((BN, ROW), lambda i: (i, 0))] * 3
    wspec = [_full(x.shape) for x in w]
    return pl.pallas_call(
        body,
        grid=(grid,),
        in_specs=nspec + wspec,
        out_specs=pl.BlockSpec((BN, ROW), lambda i: (i, 0)),
        out_shape=jax.ShapeDtypeStruct((N_N, ROW), jnp.float32),
        compiler_params=_cp(),
    )(t, p0, p1, *w)


def _out_proj(t, wh, ws_s, ws_n, b):
    def body(t_r, wh_r, wss_r, wsn_r, b_r, o_r):
        dot = functools.partial(jnp.dot, preferred_element_type=jnp.float32)
        tv = t_r[...]
        v = _vslices(tv)
        vh = [dot(x, wh_r[...]) for x in v]
        vn = jnp.sqrt(jnp.maximum(vh[0] ** 2 + vh[1] ** 2 + vh[2] ** 2, EPS))
        o_r[...] = dot(tv[:, 0:100], wss_r[...]) + dot(vn, wsn_r[...]) + b_r[...]

    grid = N_N // BN
    return pl.pallas_call(
        body,
        grid=(grid,),
        in_specs=[
            pl.BlockSpec((BN, ROW), lambda i: (i, 0)),
            _full((16, 16)), _full((100, 20)), _full((16, 20)), _full((1, 20)),
        ],
        out_specs=pl.BlockSpec((BN, 20), lambda i: (i, 0)),
        out_shape=jax.ShapeDtypeStruct((N_N, 20), jnp.float32),
        compiler_params=_cp(),
    )(t, wh, ws_s, ws_n, b)


def _msg_weights(mp, se):
    p1, p2, p3 = mp
    wh = p1['wh']
    ws = p1['ws_w']
    w = [wh[0:16], wh[16:17], wh[17:33],
         ws[0:100], ws[100:100 + se], ws[100 + se:200 + se],
         ws[200 + se:233 + se], p1['ws_b'][None, :], p1['wv']]
    for p in (p2, p3):
        ws = p['ws_w']
        w += [p['wh'], ws[0:100], ws[100:116], p['ws_b'][None, :], p['wv']]
    return w


def _ff_weights(lp):
    f1, f2 = lp['ff']
    return [lp['ln0']['g'][None, :], lp['ln0']['b'][None, :],
            f1['wh'], f1['ws_w'][0:100], f1['ws_w'][100:132],
            f1['ws_b'][None, :], f1['wv'],
            f2['wh'], f2['ws_w'][0:400], f2['ws_w'][400:432],
            f2['ws_b'][None, :], f2['wv'],
            lp['ln1']['g'][None, :], lp['ln1']['b'][None, :]]


def kernel(h_V_s, h_V_v, edge_index, h_E_s, h_E_v, seq, params):
    src = edge_index[0]
    dst = edge_index[1]
    fwd = src < dst

    def pad_idx(a):
        return jnp.pad(a, (0, E_PAD - N_E)).reshape(NW * NCH, CH)

    src2 = pad_idx(src)
    dst2 = pad_idx(dst)
    off = jnp.where(fwd, 0, N_N).astype(jnp.int32)
    ijd2 = pad_idx(src + off)
    iid2 = pad_idx(dst + off)

    z160 = jnp.zeros((N_PAD, ROW), jnp.float32)
    z16 = jnp.zeros((N_PAD, 16), jnp.float32)
    ones16 = jnp.concatenate([jnp.ones((N_E, 16), jnp.float32),
                              jnp.zeros((E_PAD - N_E, 16), jnp.float32)], 0)

    c0, c1 = _sc_scatter_add(ones16, dst2, z16, 16)

    hvt = jnp.swapaxes(h_V_v, 1, 2).reshape(N_N, 9)
    pv = params['W_v']
    ws = pv['ws_w']
    t = _node_embed(h_V_s, hvt, c0, c1, pv['wh'], ws[0:6], ws[6:22],
                    pv['ws_b'][None, :], pv['wv'],
                    params['W_v_ln']['g'][None, :],
                    params['W_v_ln']['b'][None, :])

    hes = jnp.pad(h_E_s, ((0, E_PAD - N_E), (0, 0)))
    hev = jnp.pad(h_E_v.reshape(N_E, 3), ((0, E_PAD - N_E), (0, 5)))
    pe = params['W_e']
    ws = pe['ws_w']
    ef = _edge_embed(hes, hev, ws[0:32], ws[32:33], pe['ws_b'][None, :],
                     pe['wh'], pe['wv'],
                     params['W_e_ln']['g'][None, :],
                     params['W_e_ln']['b'][None, :])

    for lp in params['enc']:
        gj, gi = _sc_gather2(t, src2, dst2, ROW, jnp.float32)
        msg = _msg_mlp(gj, gi, ef, 32, _msg_weights(lp['msg'], 32))
        p0, p1 = _sc_scatter_add(msg, dst2, z160, ROW)
        t = _node_update(t, p0, p1, _ff_weights(lp))
    ar = t

    seq_tab = jnp.tile(seq[:, None], (1, 16))
    seq_g = _sc_gather1(seq_tab, src2, 16, jnp.int32)
    ws_pad = jnp.pad(params['W_s'], ((0, 1), (0, 12)))
    hs_tab = jnp.tile(ws_pad, (16, 1))
    fwd_pad = jnp.pad(fwd, (0, E_PAD - N_E))
    idx_hs = jnp.where(fwd_pad, seq_g[:, 0], 20)
    idx_hs = idx_hs + 21 * (jnp.arange(E_PAD, dtype=jnp.int32) % 16)
    hsg = _sc_gather1(hs_tab, idx_hs.astype(jnp.int32).reshape(NW * NCH, CH),
                      32, jnp.float32)
    ef2 = _efdec_build(ef, hsg)

    for lp in params['dec']:
        tabd = jnp.concatenate([t, ar], 0)
        gj, gi = _sc_gather2(tabd, ijd2, iid2, ROW, jnp.float32)
        msg = _msg_mlp(gj, gi, ef2, 52, _msg_weights(lp['msg'], 52))
        p0, p1 = _sc_scatter_add(msg, dst2, z160, ROW)
        t = _node_update(t, p0, p1, _ff_weights(lp))

    po = params['W_out']
    ws = po['ws_w']
    return _out_proj(t, po['wh'], ws[0:100], ws[100:116], po['ws_b'][None, :])


# speedup vs baseline: 7.0769x; 1.0375x over previous
import functools

import jax
import jax.numpy as jnp
from jax import lax
from jax.experimental import pallas as pl
from jax.experimental.pallas import tpu as pltpu
from jax.experimental.pallas import tpu_sc as plsc

N_N = 10000
N_E = 160000
ROW = 160
NC, NS = 2, 16
NW = NC * NS
CH = 128
NCH = 40
E_PAD = NW * NCH * CH
N_PAD = 10240
BE = 2048
BN = 2000
EPS = 1e-8


def _sc_mesh():
    return plsc.VectorSubcoreMesh(core_axis_name="c", subcore_axis_name="s")


def _sc_cp():
    return pltpu.CompilerParams(use_tc_tiling_on_sc=False)


def _sc_gather2(tab, idxj2, idxi2, D, dtype):
    out_t = jax.ShapeDtypeStruct((E_PAD, D), dtype)

    @functools.partial(
        pl.kernel,
        out_type=[out_t, out_t],
        mesh=_sc_mesh(),
        compiler_params=_sc_cp(),
        scratch_types=[
            pltpu.VMEM((NCH, CH), jnp.int32),
            pltpu.VMEM((NCH, CH), jnp.int32),
            pltpu.VMEM((CH, D), dtype),
            pltpu.VMEM((CH, D), dtype),
            pltpu.VMEM((CH, D), dtype),
            pltpu.VMEM((CH, D), dtype),
            pltpu.SemaphoreType.DMA,
            pltpu.SemaphoreType.DMA,
            pltpu.SemaphoreType.DMA,
            pltpu.SemaphoreType.DMA,
        ],
    )
    def k(tab_h, ij_h, ii_h, oj_h, oi_h, ijv, iiv, bj0, bi0, bj1, bi1,
          sj0, si0, sj1, si1):
        wid = lax.axis_index("s") * NC + lax.axis_index("c")
        c0 = wid * NCH
        pltpu.sync_copy(ij_h.at[pl.ds(c0, NCH)], ijv)
        pltpu.sync_copy(ii_h.at[pl.ds(c0, NCH)], iiv)

        def issue(t, bj, bi, sj, si):
            pltpu.async_copy(tab_h.at[ijv.at[t]], bj, sj)
            pltpu.async_copy(tab_h.at[iiv.at[t]], bi, si)

        def drain_write(t, bj, bi, sj, si):
            pltpu.make_async_copy(tab_h.at[ijv.at[0]], bj, sj).wait()
            base = (c0 + t) * CH
            pltpu.sync_copy(bj, oj_h.at[pl.ds(base, CH)])
            pltpu.make_async_copy(tab_h.at[iiv.at[0]], bi, si).wait()
            pltpu.sync_copy(bi, oi_h.at[pl.ds(base, CH)])

        issue(0, bj0, bi0, sj0, si0)

        @pl.loop(0, NCH // 2)
        def _(u):
            t0 = 2 * u
            issue(t0 + 1, bj1, bi1, sj1, si1)
            drain_write(t0, bj0, bi0, sj0, si0)

            @pl.when(u < NCH // 2 - 1)
            def _():
                issue(t0 + 2, bj0, bi0, sj0, si0)

            drain_write(t0 + 1, bj1, bi1, sj1, si1)

    return k(tab, idxj2, idxi2)


def _sc_gather1(tab, idx2, D, dtype):
    out_t = jax.ShapeDtypeStruct((E_PAD, D), dtype)

    @functools.partial(
        pl.kernel,
        out_type=out_t,
        mesh=_sc_mesh(),
        compiler_params=_sc_cp(),
        scratch_types=[
            pltpu.VMEM((NCH, CH), jnp.int32),
            pltpu.VMEM((CH, D), dtype),
            pltpu.SemaphoreType.DMA,
        ],
    )
    def k(tab_h, ix_h, o_h, ixv, buf, sem):
        wid = lax.axis_index("s") * NC + lax.axis_index("c")
        c0 = wid * NCH
        pltpu.sync_copy(ix_h.at[pl.ds(c0, NCH)], ixv)

        @pl.loop(0, NCH)
        def _(t):
            pltpu.async_copy(tab_h.at[ixv.at[t]], buf, sem).wait()
            pltpu.sync_copy(buf, o_h.at[pl.ds((c0 + t) * CH, CH)])

    return k(tab, idx2)


CH_S = 64
NCH_S = 80


def _sc_scatter_add(vals, idx2, zeros, D):
    out_t = jax.ShapeDtypeStruct((N_PAD, D), jnp.float32)
    rows_per_sub = N_PAD // NS

    @functools.partial(
        pl.kernel,
        out_type=[out_t, out_t],
        mesh=_sc_mesh(),
        compiler_params=_sc_cp(),
        scratch_types=[
            pltpu.VMEM((NCH_S, CH_S), jnp.int32),
            pltpu.VMEM((CH_S, D), jnp.float32),
            pltpu.VMEM((CH_S, D), jnp.float32),
            pltpu.VMEM_SHARED((N_PAD, D), jnp.float32),
            pltpu.SemaphoreType.DMA,
            pltpu.SemaphoreType.DMA,
        ],
    )
    def k(v_h, ix_h, z_h, o0_h, o1_h, ixv, b0, b1, acc, s0, s1):
        cid = lax.axis_index("c")
        sid = lax.axis_index("s")
        wid = sid * NC + cid
        c0 = wid * NCH_S
        pltpu.sync_copy(ix_h.at[pl.ds(c0, NCH_S)], ixv)
        r0 = sid * rows_per_sub
        pltpu.sync_copy(z_h.at[pl.ds(r0, rows_per_sub)],
                        acc.at[pl.ds(r0, rows_per_sub)])
        plsc.subcore_barrier()

        def load(t, b, s):
            pltpu.async_copy(v_h.at[pl.ds((c0 + t) * CH_S, CH_S)], b, s)

        def add(t, b, s):
            pltpu.make_async_copy(v_h.at[pl.ds(c0 * CH_S, CH_S)], b, s).wait()
            pltpu.sync_copy(b, acc.at[ixv.at[t]], add=True)

        load(0, b0, s0)

        @pl.loop(0, NCH_S // 2)
        def _(u):
            t0 = 2 * u
            load(t0 + 1, b1, s1)
            add(t0, b0, s0)

            @pl.when(u < NCH_S // 2 - 1)
            def _():
                load(t0 + 2, b0, s0)

            add(t0 + 1, b1, s1)

        plsc.subcore_barrier()

        @pl.when(cid == 0)
        def _():
            pltpu.sync_copy(acc.at[pl.ds(r0, rows_per_sub)],
                            o0_h.at[pl.ds(r0, rows_per_sub)])

        @pl.when(cid == 1)
        def _():
            pltpu.sync_copy(acc.at[pl.ds(r0, rows_per_sub)],
                            o1_h.at[pl.ds(r0, rows_per_sub)])

    return k(vals, idx2, zeros)


def _full(shape):
    return pl.BlockSpec(shape, lambda i: (0,) * len(shape))


def _cp():
    return pltpu.CompilerParams(dimension_semantics=("arbitrary",))


def _vslices(t):
    return [t[:, 100 + 16 * c:116 + 16 * c] for c in range(3)]


def _vec_gate(vo):
    nrm = jnp.sqrt(jnp.maximum(vo[0] * vo[0] + vo[1] * vo[1] + vo[2] * vo[2],
                               EPS))
    g = jax.nn.sigmoid(nrm)
    return [x * g for x in vo]


def _scalar_ln(s, g, b):
    mu = jnp.mean(s, axis=-1, keepdims=True)
    var = jnp.mean(jnp.square(s - mu), axis=-1, keepdims=True)
    return (s - mu) / jnp.sqrt(var + 1e-5) * g + b


def _vector_ln(v):
    nsq = jnp.maximum(v[0] * v[0] + v[1] * v[1] + v[2] * v[2], EPS)
    vn = jnp.sqrt(jnp.mean(nsq, axis=-1, keepdims=True))
    return [x / vn for x in v]


def _store_packed(o_ref, s, v, extra):
    o_ref[:, 0:100] = s
    for c in range(3):
        o_ref[:, 100 + 16 * c:116 + 16 * c] = v[c]
    o_ref[:, 148:160] = extra


def _node_embed(h_V_s, hvt, p0c, p1c, wh, ws_s, ws_n, b, wv, g, bb):
    def body(hvs_r, hvt_r, p0_r, p1_r, wh_r, wss_r, wsn_r, b_r, wv_r, g_r,
             bb_r, o_r):
        hvs, hvt_ = hvs_r[...], hvt_r[...]
        vh = [jnp.dot(hvt_[:, 3 * c:3 * c + 3], wh_r[...],
                      preferred_element_type=jnp.float32) for c in range(3)]
        vn = jnp.sqrt(jnp.maximum(vh[0] ** 2 + vh[1] ** 2 + vh[2] ** 2, EPS))
        s = (jnp.dot(hvs, wss_r[...], preferred_element_type=jnp.float32)
             + jnp.dot(vn, wsn_r[...], preferred_element_type=jnp.float32)
             + b_r[...])
        v = [jnp.dot(x, wv_r[...], preferred_element_type=jnp.float32)
             for x in vh]
        s = _scalar_ln(s, g_r[...], bb_r[...])
        v = _vector_ln(v)
        cnt = p0_r[:, 0:1] + p1_r[:, 0:1]
        cinv = 1.0 / jnp.maximum(cnt, 1.0)
        extra = jnp.concatenate(
            [cinv, jnp.zeros((cinv.shape[0], 11), jnp.float32)], axis=1)
        _store_packed(o_r, s, v, extra)

    grid = N_N // BN
    return pl.pallas_call(
        body,
        grid=(grid,),
        in_specs=[
            pl.BlockSpec((BN, 6), lambda i: (i, 0)),
            pl.BlockSpec((BN, 9), lambda i: (i, 0)),
            pl.BlockSpec((BN, 16), lambda i: (i, 0)),
            pl.BlockSpec((BN, 16), lambda i: (i, 0)),
            _full((3, 16)), _full((6, 100)), _full((16, 100)), _full((1, 100)),
            _full((16, 16)), _full((1, 100)), _full((1, 100)),
        ],
        out_specs=pl.BlockSpec((BN, ROW), lambda i: (i, 0)),
        out_shape=jax.ShapeDtypeStruct((N_N, ROW), jnp.float32),
        compiler_params=_cp(),
    )(h_V_s, hvt, p0c, p1c, wh, ws_s, ws_n, b, wv, g, bb)


def _edge_embed(hes, hev, ws_s, ws_n, b, wh00, wv00, g, bb):
    def body(hes_r, hev_r, wss_r, wsn_r, b_r, wh_r, wv_r, g_r, bb_r, o_r):
        es = hes_r[...]
        ev = [hev_r[:, c:c + 1] for c in range(3)]
        wh = wh_r[0, 0]
        vh = [x * wh for x in ev]
        vn = jnp.sqrt(jnp.maximum(vh[0] ** 2 + vh[1] ** 2 + vh[2] ** 2, EPS))
        s = (jnp.dot(es, wss_r[...], preferred_element_type=jnp.float32)
             + vn * wsn_r[...] + b_r[...])
        v = [x * wv_r[0, 0] for x in vh]
        s = _scalar_ln(s, g_r[...], bb_r[...])
        v = _vector_ln(v)
        o_r[:, 0:32] = s
        for c in range(3):
            o_r[:, 32 + c:33 + c] = v[c]
        o_r[:, 35:64] = jnp.zeros((s.shape[0], 29), jnp.float32)

    grid = E_PAD // BE
    return pl.pallas_call(
        body,
        grid=(grid,),
        in_specs=[
            pl.BlockSpec((BE, 32), lambda i: (i, 0)),
            pl.BlockSpec((BE, 8), lambda i: (i, 0)),
            _full((32, 32)), _full((1, 32)), _full((1, 32)),
            _full((1, 1)), _full((1, 1)), _full((1, 32)), _full((1, 32)),
        ],
        out_specs=pl.BlockSpec((BE, 64), lambda i: (i, 0)),
        out_shape=jax.ShapeDtypeStruct((E_PAD, 64), jnp.float32),
        compiler_params=_cp(),
    )(hes, hev, ws_s, ws_n, b, wh00, wv00, g, bb)


def _efdec_build(ef, hsg):
    def body(ef_r, hs_r, o_r):
        o_r[:, 0:32] = ef_r[:, 0:32]
        o_r[:, 32:52] = hs_r[:, 0:20]
        o_r[:, 52:55] = ef_r[:, 32:35]
        o_r[:, 55:64] = jnp.zeros((ef_r.shape[0], 9), jnp.float32)

    grid = E_PAD // BE
    return pl.pallas_call(
        body,
        grid=(grid,),
        in_specs=[pl.BlockSpec((BE, 64), lambda i: (i, 0)),
                  pl.BlockSpec((BE, 32), lambda i: (i, 0))],
        out_specs=pl.BlockSpec((BE, 64), lambda i: (i, 0)),
        out_shape=jax.ShapeDtypeStruct((E_PAD, 64), jnp.float32),
        compiler_params=_cp(),
    )(ef, hsg)


def _msg_mlp(gj, gi, ef, se, w):
    def body(gj_r, gi_r, ef_r, whj_r, whe_r, whi_r, w1j_r, w1e_r, w1i_r,
             w1n_r, b1_r, wv1_r, wh2_r, w2s_r, w2n_r, b2_r, wv2_r, wh3_r,
             w3s_r, w3n_r, b3_r, wv3_r, o_r):
        bid = pl.program_id(0)
        gjv, giv = gj_r[...], gi_r[...]
        efv = ef_r[...]
        gjs, gis = gjv[:, 0:100], giv[:, 0:100]
        es = efv[:, 0:se]

        dot = functools.partial(jnp.dot, preferred_element_type=jnp.float32)
        vh = []
        for c in range(3):
            mj = gjv[:, 100 + 16 * c:116 + 16 * c]
            mi = giv[:, 100 + 16 * c:116 + 16 * c]
            ev = efv[:, se + c:se + c + 1]
            vh.append(dot(mj, whj_r[...]) + ev * whe_r[...]
                      + dot(mi, whi_r[...]))
        vn = jnp.sqrt(jnp.maximum(vh[0] ** 2 + vh[1] ** 2 + vh[2] ** 2, EPS))
        s = (dot(gjs, w1j_r[...]) + dot(es, w1e_r[...]) + dot(gis, w1i_r[...])
             + dot(vn, w1n_r[...]) + b1_r[...])
        s = jnp.maximum(s, 0.0)
        v = _vec_gate([dot(x, wv1_r[...]) for x in vh])

        vh = [dot(x, wh2_r[...]) for x in v]
        vn = jnp.sqrt(jnp.maximum(vh[0] ** 2 + vh[1] ** 2 + vh[2] ** 2, EPS))
        s = jnp.maximum(dot(s, w2s_r[...]) + dot(vn, w2n_r[...]) + b2_r[...],
                        0.0)
        v = _vec_gate([dot(x, wv2_r[...]) for x in vh])

        vh = [dot(x, wh3_r[...]) for x in v]
        vn = jnp.sqrt(jnp.maximum(vh[0] ** 2 + vh[1] ** 2 + vh[2] ** 2, EPS))
        s = dot(s, w3s_r[...]) + dot(vn, w3n_r[...]) + b3_r[...]
        v = [dot(x, wv3_r[...]) for x in vh]

        rows = jax.lax.broadcasted_iota(jnp.int32, (gjv.shape[0], 1), 0)
        valid = (bid * BE + rows) < N_E
        ci = jnp.where(valid, giv[:, 148:149], 0.0)
        s = s * ci
        v = [x * ci for x in v]
        _store_packed(o_r, s, v, jnp.zeros((gjv.shape[0], 12), jnp.float32))

    grid = E_PAD // BE
    espec = [
        pl.BlockSpec((BE, ROW), lambda i: (i, 0)),
        pl.BlockSpec((BE, ROW), lambda i: (i, 0)),
        pl.BlockSpec((BE, 64), lambda i: (i, 0)),
    ]
    wspec = [_full(x.shape) for x in w]
    return pl.pallas_call(
        body,
        grid=(grid,),
        in_specs=espec + wspec,
        out_specs=pl.BlockSpec((BE, ROW), lambda i: (i, 0)),
        out_shape=jax.ShapeDtypeStruct((E_PAD, ROW), jnp.float32),
        compiler_params=_cp(),
    )(gj, gi, ef, *w)


def _node_update(t, p0, p1, w):
    def body(t_r, p0_r, p1_r, g0_r, b0_r, f1wh_r, f1ws_r, f1wn_r, f1b_r,
             f1wv_r, f2wh_r, f2ws_r, f2wn_r, f2b_r, f2wv_r, g1_r, b1_r, o_r):
        dot = functools.partial(jnp.dot, preferred_element_type=jnp.float32)
        tv = t_r[...]
        d0, d1 = p0_r[...], p1_r[...]
        dsum = d0 + d1
        s = tv[:, 0:100] + dsum[:, 0:100]
        v = [a + b for a, b in zip(_vslices(tv), _vslices(dsum))]
        s = _scalar_ln(s, g0_r[...], b0_r[...])
        v = _vector_ln(v)

        vh = [dot(x, f1wh_r[...]) for x in v]
        vn = jnp.sqrt(jnp.maximum(vh[0] ** 2 + vh[1] ** 2 + vh[2] ** 2, EPS))
        fs = jnp.maximum(dot(s, f1ws_r[...]) + dot(vn, f1wn_r[...])
                         + f1b_r[...], 0.0)
        fv = _vec_gate([dot(x, f1wv_r[...]) for x in vh])

        vh = [dot(x, f2wh_r[...]) for x in fv]
        vn = jnp.sqrt(jnp.maximum(vh[0] ** 2 + vh[1] ** 2 + vh[2] ** 2, EPS))
        fs = dot(fs, f2ws_r[...]) + dot(vn, f2wn_r[...]) + f2b_r[...]
        fv = [dot(x, f2wv_r[...]) for x in vh]

        s = s + fs
        v = [a + b for a, b in zip(v, fv)]
        s = _scalar_ln(s, g1_r[...], b1_r[...])
        v = _vector_ln(v)
        extra = jnp.concatenate(
            [tv[:, 148:149], jnp.zeros((tv.shape[0], 11), jnp.float32)],
            axis=1)
        _store_packed(o_r, s, v, extra)

    grid = N_N // BN
    nspec = [pl.BlockSpec((BN, ROW), lambda i: (i, 0))] * 3
    wspec = [_full(x.shape) for x in w]
    return pl.pallas_call(
        body,
        grid=(grid,),
        in_specs=nspec + wspec,
        out_specs=pl.BlockSpec((BN, ROW), lambda i: (i, 0)),
        out_shape=jax.ShapeDtypeStruct((N_N, ROW), jnp.float32),
        compiler_params=_cp(),
    )(t, p0, p1, *w)


def _out_proj(t, wh, ws_s, ws_n, b):
    def body(t_r, wh_r, wss_r, wsn_r, b_r, o_r):
        dot = functools.partial(jnp.dot, preferred_element_type=jnp.float32)
        tv = t_r[...]
        v = _vslices(tv)
        vh = [dot(x, wh_r[...]) for x in v]
        vn = jnp.sqrt(jnp.maximum(vh[0] ** 2 + vh[1] ** 2 + vh[2] ** 2, EPS))
        o_r[...] = dot(tv[:, 0:100], wss_r[...]) + dot(vn, wsn_r[...]) + b_r[...]

    grid = N_N // BN
    return pl.pallas_call(
        body,
        grid=(grid,),
        in_specs=[
            pl.BlockSpec((BN, ROW), lambda i: (i, 0)),
            _full((16, 16)), _full((100, 20)), _full((16, 20)), _full((1, 20)),
        ],
        out_specs=pl.BlockSpec((BN, 20), lambda i: (i, 0)),
        out_shape=jax.ShapeDtypeStruct((N_N, 20), jnp.float32),
        compiler_params=_cp(),
    )(t, wh, ws_s, ws_n, b)


def _msg_weights(mp, se):
    p1, p2, p3 = mp
    wh = p1['wh']
    ws = p1['ws_w']
    w = [wh[0:16], wh[16:17], wh[17:33],
         ws[0:100], ws[100:100 + se], ws[100 + se:200 + se],
         ws[200 + se:233 + se], p1['ws_b'][None, :], p1['wv']]
    for p in (p2, p3):
        ws = p['ws_w']
        w += [p['wh'], ws[0:100], ws[100:116], p['ws_b'][None, :], p['wv']]
    return w


def _ff_weights(lp):
    f1, f2 = lp['ff']
    return [lp['ln0']['g'][None, :], lp['ln0']['b'][None, :],
            f1['wh'], f1['ws_w'][0:100], f1['ws_w'][100:132],
            f1['ws_b'][None, :], f1['wv'],
            f2['wh'], f2['ws_w'][0:400], f2['ws_w'][400:432],
            f2['ws_b'][None, :], f2['wv'],
            lp['ln1']['g'][None, :], lp['ln1']['b'][None, :]]


def kernel(h_V_s, h_V_v, edge_index, h_E_s, h_E_v, seq, params):
    src = edge_index[0]
    dst = edge_index[1]
    fwd = src < dst

    def pad_idx(a):
        return jnp.pad(a, (0, E_PAD - N_E)).reshape(NW * NCH, CH)

    src2 = pad_idx(src)
    dst2 = pad_idx(dst)
    dst2s = jnp.pad(dst, (0, E_PAD - N_E)).reshape(NW * NCH_S, CH_S)
    off = jnp.where(fwd, 0, N_N).astype(jnp.int32)
    ijd2 = pad_idx(src + off)
    iid2 = pad_idx(dst + off)

    z160 = jnp.zeros((N_PAD, ROW), jnp.float32)
    z16 = jnp.zeros((N_PAD, 16), jnp.float32)
    ones16 = jnp.concatenate([jnp.ones((N_E, 16), jnp.float32),
                              jnp.zeros((E_PAD - N_E, 16), jnp.float32)], 0)

    c0, c1 = _sc_scatter_add(ones16, dst2s, z16, 16)

    hvt = jnp.swapaxes(h_V_v, 1, 2).reshape(N_N, 9)
    pv = params['W_v']
    ws = pv['ws_w']
    t = _node_embed(h_V_s, hvt, c0, c1, pv['wh'], ws[0:6], ws[6:22],
                    pv['ws_b'][None, :], pv['wv'],
                    params['W_v_ln']['g'][None, :],
                    params['W_v_ln']['b'][None, :])

    hes = jnp.pad(h_E_s, ((0, E_PAD - N_E), (0, 0)))
    hev = jnp.pad(h_E_v.reshape(N_E, 3), ((0, E_PAD - N_E), (0, 5)))
    pe = params['W_e']
    ws = pe['ws_w']
    ef = _edge_embed(hes, hev, ws[0:32], ws[32:33], pe['ws_b'][None, :],
                     pe['wh'], pe['wv'],
                     params['W_e_ln']['g'][None, :],
                     params['W_e_ln']['b'][None, :])

    for lp in params['enc']:
        gj, gi = _sc_gather2(t, src2, dst2, ROW, jnp.float32)
        msg = _msg_mlp(gj, gi, ef, 32, _msg_weights(lp['msg'], 32))
        p0, p1 = _sc_scatter_add(msg, dst2s, z160, ROW)
        t = _node_update(t, p0, p1, _ff_weights(lp))
    ar = t

    seq_tab = jnp.tile(seq[:, None], (1, 16))
    seq_g = _sc_gather1(seq_tab, src2, 16, jnp.int32)
    ws_pad = jnp.pad(params['W_s'], ((0, 1), (0, 12)))
    hs_tab = jnp.tile(ws_pad, (16, 1))
    fwd_pad = jnp.pad(fwd, (0, E_PAD - N_E))
    idx_hs = jnp.where(fwd_pad, seq_g[:, 0], 20)
    idx_hs = idx_hs + 21 * (jnp.arange(E_PAD, dtype=jnp.int32) % 16)
    hsg = _sc_gather1(hs_tab, idx_hs.astype(jnp.int32).reshape(NW * NCH, CH),
                      32, jnp.float32)
    ef2 = _efdec_build(ef, hsg)

    for lp in params['dec']:
        tabd = jnp.concatenate([t, ar], 0)
        gj, gi = _sc_gather2(tabd, ijd2, iid2, ROW, jnp.float32)
        msg = _msg_mlp(gj, gi, ef2, 52, _msg_weights(lp['msg'], 52))
        p0, p1 = _sc_scatter_add(msg, dst2s, z160, ROW)
        t = _node_update(t, p0, p1, _ff_weights(lp))

    po = params['W_out']
    ws = po['ws_w']
    return _out_proj(t, po['wh'], ws[0:100], ws[100:116], po['ws_b'][None, :])


# speedup vs baseline: 7.3771x; 1.0424x over previous
import functools

import jax
import jax.numpy as jnp
from jax import lax
from jax.experimental import pallas as pl
from jax.experimental.pallas import tpu as pltpu
from jax.experimental.pallas import tpu_sc as plsc

N_N = 10000
N_E = 160000
ROW = 160
NC, NS = 2, 16
NW = NC * NS
CH = 128
NCH = 40
E_PAD = NW * NCH * CH
N_PAD = 10240
BE = 2048
BN = 2000
EPS = 1e-8


def _sc_mesh():
    return plsc.VectorSubcoreMesh(core_axis_name="c", subcore_axis_name="s")


def _sc_cp():
    return pltpu.CompilerParams(use_tc_tiling_on_sc=False)


def _sc_gather2(tab, idxj2, idxi2, D, dtype):
    out_t = jax.ShapeDtypeStruct((E_PAD, D), dtype)

    @functools.partial(
        pl.kernel,
        out_type=[out_t, out_t],
        mesh=_sc_mesh(),
        compiler_params=_sc_cp(),
        scratch_types=[
            pltpu.VMEM((NCH, CH), jnp.int32),
            pltpu.VMEM((NCH, CH), jnp.int32),
            pltpu.VMEM((CH, D), dtype),
            pltpu.VMEM((CH, D), dtype),
            pltpu.VMEM((CH, D), dtype),
            pltpu.VMEM((CH, D), dtype),
            pltpu.SemaphoreType.DMA,
            pltpu.SemaphoreType.DMA,
            pltpu.SemaphoreType.DMA,
            pltpu.SemaphoreType.DMA,
        ],
    )
    def k(tab_h, ij_h, ii_h, oj_h, oi_h, ijv, iiv, bj0, bi0, bj1, bi1,
          sj0, si0, sj1, si1):
        wid = lax.axis_index("s") * NC + lax.axis_index("c")
        c0 = wid * NCH
        pltpu.sync_copy(ij_h.at[pl.ds(c0, NCH)], ijv)
        pltpu.sync_copy(ii_h.at[pl.ds(c0, NCH)], iiv)

        def issue(t, bj, bi, sj, si):
            pltpu.async_copy(tab_h.at[ijv.at[t]], bj, sj)
            pltpu.async_copy(tab_h.at[iiv.at[t]], bi, si)

        def drain_write(t, bj, bi, sj, si):
            pltpu.make_async_copy(tab_h.at[ijv.at[0]], bj, sj).wait()
            base = (c0 + t) * CH
            pltpu.sync_copy(bj, oj_h.at[pl.ds(base, CH)])
            pltpu.make_async_copy(tab_h.at[iiv.at[0]], bi, si).wait()
            pltpu.sync_copy(bi, oi_h.at[pl.ds(base, CH)])

        issue(0, bj0, bi0, sj0, si0)

        @pl.loop(0, NCH // 2)
        def _(u):
            t0 = 2 * u
            issue(t0 + 1, bj1, bi1, sj1, si1)
            drain_write(t0, bj0, bi0, sj0, si0)

            @pl.when(u < NCH // 2 - 1)
            def _():
                issue(t0 + 2, bj0, bi0, sj0, si0)

            drain_write(t0 + 1, bj1, bi1, sj1, si1)

    return k(tab, idxj2, idxi2)


def _sc_gather1(tab, idx2, D, dtype):
    out_t = jax.ShapeDtypeStruct((E_PAD, D), dtype)

    @functools.partial(
        pl.kernel,
        out_type=out_t,
        mesh=_sc_mesh(),
        compiler_params=_sc_cp(),
        scratch_types=[
            pltpu.VMEM((NCH, CH), jnp.int32),
            pltpu.VMEM((CH, D), dtype),
            pltpu.SemaphoreType.DMA,
        ],
    )
    def k(tab_h, ix_h, o_h, ixv, buf, sem):
        wid = lax.axis_index("s") * NC + lax.axis_index("c")
        c0 = wid * NCH
        pltpu.sync_copy(ix_h.at[pl.ds(c0, NCH)], ixv)

        @pl.loop(0, NCH)
        def _(t):
            pltpu.async_copy(tab_h.at[ixv.at[t]], buf, sem).wait()
            pltpu.sync_copy(buf, o_h.at[pl.ds((c0 + t) * CH, CH)])

    return k(tab, idx2)


CH_S = 64
NCH_S = 80


def _sc_scatter_add(vals, idx2, zeros, D):
    out_t = jax.ShapeDtypeStruct((N_PAD, D), jnp.float32)
    rows_per_sub = N_PAD // NS

    @functools.partial(
        pl.kernel,
        out_type=[out_t, out_t],
        mesh=_sc_mesh(),
        compiler_params=_sc_cp(),
        scratch_types=[
            pltpu.VMEM((NCH_S, CH_S), jnp.int32),
            pltpu.VMEM((CH_S, D), jnp.float32),
            pltpu.VMEM((CH_S, D), jnp.float32),
            pltpu.VMEM_SHARED((N_PAD, D), jnp.float32),
            pltpu.SemaphoreType.DMA,
            pltpu.SemaphoreType.DMA,
        ],
    )
    def k(v_h, ix_h, z_h, o0_h, o1_h, ixv, b0, b1, acc, s0, s1):
        cid = lax.axis_index("c")
        sid = lax.axis_index("s")
        wid = sid * NC + cid
        c0 = wid * NCH_S
        pltpu.sync_copy(ix_h.at[pl.ds(c0, NCH_S)], ixv)
        r0 = sid * rows_per_sub
        pltpu.sync_copy(z_h.at[pl.ds(r0, rows_per_sub)],
                        acc.at[pl.ds(r0, rows_per_sub)])
        plsc.subcore_barrier()

        def load(t, b, s):
            pltpu.async_copy(v_h.at[pl.ds((c0 + t) * CH_S, CH_S)], b, s)

        def add(t, b, s):
            pltpu.make_async_copy(v_h.at[pl.ds(c0 * CH_S, CH_S)], b, s).wait()
            pltpu.sync_copy(b, acc.at[ixv.at[t]], add=True)

        load(0, b0, s0)

        @pl.loop(0, NCH_S // 2)
        def _(u):
            t0 = 2 * u
            load(t0 + 1, b1, s1)
            add(t0, b0, s0)

            @pl.when(u < NCH_S // 2 - 1)
            def _():
                load(t0 + 2, b0, s0)

            add(t0 + 1, b1, s1)

        plsc.subcore_barrier()

        @pl.when(cid == 0)
        def _():
            pltpu.sync_copy(acc.at[pl.ds(r0, rows_per_sub)],
                            o0_h.at[pl.ds(r0, rows_per_sub)])

        @pl.when(cid == 1)
        def _():
            pltpu.sync_copy(acc.at[pl.ds(r0, rows_per_sub)],
                            o1_h.at[pl.ds(r0, rows_per_sub)])

    return k(vals, idx2, zeros)


def _full(shape):
    return pl.BlockSpec(shape, lambda i: (0,) * len(shape))


def _cp():
    return pltpu.CompilerParams(dimension_semantics=("arbitrary",))


def _vslices(t):
    return [t[:, 100 + 16 * c:116 + 16 * c] for c in range(3)]


def _vec_gate(vo):
    nrm = jnp.sqrt(jnp.maximum(vo[0] * vo[0] + vo[1] * vo[1] + vo[2] * vo[2],
                               EPS))
    g = jax.nn.sigmoid(nrm)
    return [x * g for x in vo]


def _scalar_ln(s, g, b):
    mu = jnp.mean(s, axis=-1, keepdims=True)
    var = jnp.mean(jnp.square(s - mu), axis=-1, keepdims=True)
    return (s - mu) / jnp.sqrt(var + 1e-5) * g + b


def _vector_ln(v):
    nsq = jnp.maximum(v[0] * v[0] + v[1] * v[1] + v[2] * v[2], EPS)
    vn = jnp.sqrt(jnp.mean(nsq, axis=-1, keepdims=True))
    return [x / vn for x in v]


def _store_packed(o_ref, s, v, extra):
    dt = o_ref.dtype
    o_ref[:, 0:100] = s.astype(dt)
    for c in range(3):
        o_ref[:, 100 + 16 * c:116 + 16 * c] = v[c].astype(dt)
    o_ref[:, 148:160] = extra.astype(dt)


def _node_embed(h_V_s, hvt, wh, ws_s, ws_n, b, wv, g, bb):
    def body(hvs_r, hvt_r, wh_r, wss_r, wsn_r, b_r, wv_r, g_r,
             bb_r, o_r):
        hvs, hvt_ = hvs_r[...], hvt_r[...]
        vh = [jnp.dot(hvt_[:, 3 * c:3 * c + 3], wh_r[...],
                      preferred_element_type=jnp.float32) for c in range(3)]
        vn = jnp.sqrt(jnp.maximum(vh[0] ** 2 + vh[1] ** 2 + vh[2] ** 2, EPS))
        s = (jnp.dot(hvs, wss_r[...], preferred_element_type=jnp.float32)
             + jnp.dot(vn, wsn_r[...], preferred_element_type=jnp.float32)
             + b_r[...])
        v = [jnp.dot(x, wv_r[...], preferred_element_type=jnp.float32)
             for x in vh]
        s = _scalar_ln(s, g_r[...], bb_r[...])
        v = _vector_ln(v)
        _store_packed(o_r, s, v, jnp.zeros((s.shape[0], 12), jnp.float32))

    grid = N_N // BN
    return pl.pallas_call(
        body,
        grid=(grid,),
        in_specs=[
            pl.BlockSpec((BN, 6), lambda i: (i, 0)),
            pl.BlockSpec((BN, 9), lambda i: (i, 0)),
            _full((3, 16)), _full((6, 100)), _full((16, 100)), _full((1, 100)),
            _full((16, 16)), _full((1, 100)), _full((1, 100)),
        ],
        out_specs=pl.BlockSpec((BN, ROW), lambda i: (i, 0)),
        out_shape=jax.ShapeDtypeStruct((N_N, ROW), jnp.bfloat16),
        compiler_params=_cp(),
    )(h_V_s, hvt, wh, ws_s, ws_n, b, wv, g, bb)


def _edge_embed(hes, hev, ws_s, ws_n, b, wh00, wv00, g, bb):
    def body(hes_r, hev_r, wss_r, wsn_r, b_r, wh_r, wv_r, g_r, bb_r, o_r):
        es = hes_r[...]
        ev = [hev_r[:, c:c + 1] for c in range(3)]
        wh = wh_r[0, 0]
        vh = [x * wh for x in ev]
        vn = jnp.sqrt(jnp.maximum(vh[0] ** 2 + vh[1] ** 2 + vh[2] ** 2, EPS))
        s = (jnp.dot(es, wss_r[...], preferred_element_type=jnp.float32)
             + vn * wsn_r[...] + b_r[...])
        v = [x * wv_r[0, 0] for x in vh]
        s = _scalar_ln(s, g_r[...], bb_r[...])
        v = _vector_ln(v)
        o_r[:, 0:32] = s
        for c in range(3):
            o_r[:, 32 + c:33 + c] = v[c]
        o_r[:, 35:64] = jnp.zeros((s.shape[0], 29), jnp.float32)

    grid = E_PAD // BE
    return pl.pallas_call(
        body,
        grid=(grid,),
        in_specs=[
            pl.BlockSpec((BE, 32), lambda i: (i, 0)),
            pl.BlockSpec((BE, 8), lambda i: (i, 0)),
            _full((32, 32)), _full((1, 32)), _full((1, 32)),
            _full((1, 1)), _full((1, 1)), _full((1, 32)), _full((1, 32)),
        ],
        out_specs=pl.BlockSpec((BE, 64), lambda i: (i, 0)),
        out_shape=jax.ShapeDtypeStruct((E_PAD, 64), jnp.float32),
        compiler_params=_cp(),
    )(hes, hev, ws_s, ws_n, b, wh00, wv00, g, bb)


def _efdec_build(ef, hsg):
    def body(ef_r, hs_r, o_r):
        o_r[:, 0:32] = ef_r[:, 0:32]
        o_r[:, 32:52] = hs_r[:, 0:20]
        o_r[:, 52:55] = ef_r[:, 32:35]
        o_r[:, 55:64] = jnp.zeros((ef_r.shape[0], 9), jnp.float32)

    grid = E_PAD // BE
    return pl.pallas_call(
        body,
        grid=(grid,),
        in_specs=[pl.BlockSpec((BE, 64), lambda i: (i, 0)),
                  pl.BlockSpec((BE, 32), lambda i: (i, 0))],
        out_specs=pl.BlockSpec((BE, 64), lambda i: (i, 0)),
        out_shape=jax.ShapeDtypeStruct((E_PAD, 64), jnp.float32),
        compiler_params=_cp(),
    )(ef, hsg)


def _msg_mlp(gj, gi, ef, se, w):
    def body(gj_r, gi_r, ef_r, whj_r, whe_r, whi_r, w1j_r, w1e_r, w1i_r,
             w1n_r, b1_r, wv1_r, wh2_r, w2s_r, w2n_r, b2_r, wv2_r, wh3_r,
             w3s_r, w3n_r, b3_r, wv3_r, o_r):
        bid = pl.program_id(0)
        gjv = gj_r[...].astype(jnp.float32)
        giv = gi_r[...].astype(jnp.float32)
        efv = ef_r[...]
        gjs, gis = gjv[:, 0:100], giv[:, 0:100]
        es = efv[:, 0:se]

        dot = functools.partial(jnp.dot, preferred_element_type=jnp.float32)
        vh = []
        for c in range(3):
            mj = gjv[:, 100 + 16 * c:116 + 16 * c]
            mi = giv[:, 100 + 16 * c:116 + 16 * c]
            ev = efv[:, se + c:se + c + 1]
            vh.append(dot(mj, whj_r[...]) + ev * whe_r[...]
                      + dot(mi, whi_r[...]))
        vn = jnp.sqrt(jnp.maximum(vh[0] ** 2 + vh[1] ** 2 + vh[2] ** 2, EPS))
        s = (dot(gjs, w1j_r[...]) + dot(es, w1e_r[...]) + dot(gis, w1i_r[...])
             + dot(vn, w1n_r[...]) + b1_r[...])
        s = jnp.maximum(s, 0.0)
        v = _vec_gate([dot(x, wv1_r[...]) for x in vh])

        vh = [dot(x, wh2_r[...]) for x in v]
        vn = jnp.sqrt(jnp.maximum(vh[0] ** 2 + vh[1] ** 2 + vh[2] ** 2, EPS))
        s = jnp.maximum(dot(s, w2s_r[...]) + dot(vn, w2n_r[...]) + b2_r[...],
                        0.0)
        v = _vec_gate([dot(x, wv2_r[...]) for x in vh])

        vh = [dot(x, wh3_r[...]) for x in v]
        vn = jnp.sqrt(jnp.maximum(vh[0] ** 2 + vh[1] ** 2 + vh[2] ** 2, EPS))
        s = dot(s, w3s_r[...]) + dot(vn, w3n_r[...]) + b3_r[...]
        v = [dot(x, wv3_r[...]) for x in vh]

        rows = jax.lax.broadcasted_iota(jnp.int32, (gjv.shape[0], 1), 0)
        ci = jnp.where((bid * BE + rows) < N_E, 1.0, 0.0)
        s = s * ci
        v = [x * ci for x in v]
        _store_packed(o_r, s, v, jnp.zeros((gjv.shape[0], 12), jnp.float32))

    grid = E_PAD // BE
    espec = [
        pl.BlockSpec((BE, ROW), lambda i: (i, 0)),
        pl.BlockSpec((BE, ROW), lambda i: (i, 0)),
        pl.BlockSpec((BE, 64), lambda i: (i, 0)),
    ]
    wspec = [_full(x.shape) for x in w]
    return pl.pallas_call(
        body,
        grid=(grid,),
        in_specs=espec + wspec,
        out_specs=pl.BlockSpec((BE, ROW), lambda i: (i, 0)),
        out_shape=jax.ShapeDtypeStruct((E_PAD, ROW), jnp.float32),
        compiler_params=_cp(),
    )(gj, gi, ef, *w)


def _node_update(t, p0, p1, c0, c1, w):
    def body(t_r, p0_r, p1_r, c0_r, c1_r, g0_r, b0_r, f1wh_r, f1ws_r,
             f1wn_r, f1b_r,
             f1wv_r, f2wh_r, f2ws_r, f2wn_r, f2b_r, f2wv_r, g1_r, b1_r, o_r):
        dot = functools.partial(jnp.dot, preferred_element_type=jnp.float32)
        tv = t_r[...].astype(jnp.float32)
        cnt = c0_r[:, 0:1] + c1_r[:, 0:1]
        cinv = 1.0 / jnp.maximum(cnt, 1.0)
        dsum = (p0_r[...] + p1_r[...]) * cinv
        s = tv[:, 0:100] + dsum[:, 0:100]
        v = [a + b for a, b in zip(_vslices(tv), _vslices(dsum))]
        s = _scalar_ln(s, g0_r[...], b0_r[...])
        v = _vector_ln(v)

        vh = [dot(x, f1wh_r[...]) for x in v]
        vn = jnp.sqrt(jnp.maximum(vh[0] ** 2 + vh[1] ** 2 + vh[2] ** 2, EPS))
        fs = jnp.maximum(dot(s, f1ws_r[...]) + dot(vn, f1wn_r[...])
                         + f1b_r[...], 0.0)
        fv = _vec_gate([dot(x, f1wv_r[...]) for x in vh])

        vh = [dot(x, f2wh_r[...]) for x in fv]
        vn = jnp.sqrt(jnp.maximum(vh[0] ** 2 + vh[1] ** 2 + vh[2] ** 2, EPS))
        fs = dot(fs, f2ws_r[...]) + dot(vn, f2wn_r[...]) + f2b_r[...]
        fv = [dot(x, f2wv_r[...]) for x in vh]

        s = s + fs
        v = [a + b for a, b in zip(v, fv)]
        s = _scalar_ln(s, g1_r[...], b1_r[...])
        v = _vector_ln(v)
        _store_packed(o_r, s, v, jnp.zeros((s.shape[0], 12), jnp.float32))

    grid = N_N // BN
    nspec = [pl.BlockSpec((BN, ROW), lambda i: (i, 0))] * 3
    cspec = [pl.BlockSpec((BN, 16), lambda i: (i, 0))] * 2
    wspec = [_full(x.shape) for x in w]
    return pl.pallas_call(
        body,
        grid=(grid,),
        in_specs=nspec + cspec + wspec,
        out_specs=pl.BlockSpec((BN, ROW), lambda i: (i, 0)),
        out_shape=jax.ShapeDtypeStruct((N_N, ROW), jnp.bfloat16),
        compiler_params=_cp(),
    )(t, p0, p1, c0, c1, *w)


def _out_proj(t, wh, ws_s, ws_n, b):
    def body(t_r, wh_r, wss_r, wsn_r, b_r, o_r):
        dot = functools.partial(jnp.dot, preferred_element_type=jnp.float32)
        tv = t_r[...].astype(jnp.float32)
        v = _vslices(tv)
        vh = [dot(x, wh_r[...]) for x in v]
        vn = jnp.sqrt(jnp.maximum(vh[0] ** 2 + vh[1] ** 2 + vh[2] ** 2, EPS))
        o_r[...] = dot(tv[:, 0:100], wss_r[...]) + dot(vn, wsn_r[...]) + b_r[...]

    grid = N_N // BN
    return pl.pallas_call(
        body,
        grid=(grid,),
        in_specs=[
            pl.BlockSpec((BN, ROW), lambda i: (i, 0)),
            _full((16, 16)), _full((100, 20)), _full((16, 20)), _full((1, 20)),
        ],
        out_specs=pl.BlockSpec((BN, 20), lambda i: (i, 0)),
        out_shape=jax.ShapeDtypeStruct((N_N, 20), jnp.float32),
        compiler_params=_cp(),
    )(t, wh, ws_s, ws_n, b)


def _msg_weights(mp, se):
    p1, p2, p3 = mp
    wh = p1['wh']
    ws = p1['ws_w']
    w = [wh[0:16], wh[16:17], wh[17:33],
         ws[0:100], ws[100:100 + se], ws[100 + se:200 + se],
         ws[200 + se:233 + se], p1['ws_b'][None, :], p1['wv']]
    for p in (p2, p3):
        ws = p['ws_w']
        w += [p['wh'], ws[0:100], ws[100:116], p['ws_b'][None, :], p['wv']]
    return w


def _ff_weights(lp):
    f1, f2 = lp['ff']
    return [lp['ln0']['g'][None, :], lp['ln0']['b'][None, :],
            f1['wh'], f1['ws_w'][0:100], f1['ws_w'][100:132],
            f1['ws_b'][None, :], f1['wv'],
            f2['wh'], f2['ws_w'][0:400], f2['ws_w'][400:432],
            f2['ws_b'][None, :], f2['wv'],
            lp['ln1']['g'][None, :], lp['ln1']['b'][None, :]]


def kernel(h_V_s, h_V_v, edge_index, h_E_s, h_E_v, seq, params):
    src = edge_index[0]
    dst = edge_index[1]
    fwd = src < dst

    def pad_idx(a):
        return jnp.pad(a, (0, E_PAD - N_E)).reshape(NW * NCH, CH)

    src2 = pad_idx(src)
    dst2 = pad_idx(dst)
    dst2s = jnp.pad(dst, (0, E_PAD - N_E)).reshape(NW * NCH_S, CH_S)
    off = jnp.where(fwd, 0, N_N).astype(jnp.int32)
    ijd2 = pad_idx(src + off)
    iid2 = pad_idx(dst + off)

    z160 = jnp.zeros((N_PAD, ROW), jnp.float32)
    z16 = jnp.zeros((N_PAD, 16), jnp.float32)
    ones16 = jnp.concatenate([jnp.ones((N_E, 16), jnp.float32),
                              jnp.zeros((E_PAD - N_E, 16), jnp.float32)], 0)

    c0, c1 = _sc_scatter_add(ones16, dst2s, z16, 16)

    hvt = jnp.swapaxes(h_V_v, 1, 2).reshape(N_N, 9)
    pv = params['W_v']
    ws = pv['ws_w']
    t = _node_embed(h_V_s, hvt, pv['wh'], ws[0:6], ws[6:22],
                    pv['ws_b'][None, :], pv['wv'],
                    params['W_v_ln']['g'][None, :],
                    params['W_v_ln']['b'][None, :])

    hes = jnp.pad(h_E_s, ((0, E_PAD - N_E), (0, 0)))
    hev = jnp.pad(h_E_v.reshape(N_E, 3), ((0, E_PAD - N_E), (0, 5)))
    pe = params['W_e']
    ws = pe['ws_w']
    ef = _edge_embed(hes, hev, ws[0:32], ws[32:33], pe['ws_b'][None, :],
                     pe['wh'], pe['wv'],
                     params['W_e_ln']['g'][None, :],
                     params['W_e_ln']['b'][None, :])

    for lp in params['enc']:
        gj, gi = _sc_gather2(t, src2, dst2, ROW, jnp.bfloat16)
        msg = _msg_mlp(gj, gi, ef, 32, _msg_weights(lp['msg'], 32))
        p0, p1 = _sc_scatter_add(msg, dst2s, z160, ROW)
        t = _node_update(t, p0, p1, c0, c1, _ff_weights(lp))
    ar = t

    seq_tab = jnp.tile(seq[:, None], (1, 16))
    seq_g = _sc_gather1(seq_tab, src2, 16, jnp.int32)
    ws_pad = jnp.pad(params['W_s'], ((0, 1), (0, 12)))
    hs_tab = jnp.tile(ws_pad, (16, 1))
    fwd_pad = jnp.pad(fwd, (0, E_PAD - N_E))
    idx_hs = jnp.where(fwd_pad, seq_g[:, 0], 20)
    idx_hs = idx_hs + 21 * (jnp.arange(E_PAD, dtype=jnp.int32) % 16)
    hsg = _sc_gather1(hs_tab, idx_hs.astype(jnp.int32).reshape(NW * NCH, CH),
                      32, jnp.float32)
    ef2 = _efdec_build(ef, hsg)

    for lp in params['dec']:
        tabd = jnp.concatenate([t, ar], 0)
        gj, gi = _sc_gather2(tabd, ijd2, iid2, ROW, jnp.bfloat16)
        msg = _msg_mlp(gj, gi, ef2, 52, _msg_weights(lp['msg'], 52))
        p0, p1 = _sc_scatter_add(msg, dst2s, z160, ROW)
        t = _node_update(t, p0, p1, c0, c1, _ff_weights(lp))

    po = params['W_out']
    ws = po['ws_w']
    return _out_proj(t, po['wh'], ws[0:100], ws[100:116], po['ws_b'][None, :])
